# baseline (device time: 20412 ns/iter reference)
import jax
import jax.numpy as jnp
from jax import lax
from jax.experimental import pallas as pl
from jax.experimental.pallas import tpu as pltpu

N_DEV = 32
T = 1024
D = 256
E = 128
H = 512
E_LOCAL = E // N_DEV
TPB = T // N_DEV
CAP = 32
NSLOT = E_LOCAL * CAP


def kernel(x, router_W, route_idx, expert_W, shared_W):
    route_row = route_idx.T

    def body(x_hbm, rW_hbm, rrow_hbm, eW_hbm, sW_hbm, out_ref,
             x_vm, rW_vm, rrow_vm, eW_vm, sW_vm,
             y_buf, recv_buf, meta_vm, n_vm, meta_sm, n_sm,
             in_sems, prep_sem, send_sem, recv_sem):
        me = lax.axis_index("i")

        fetches = [
            pltpu.make_async_copy(src, dst, in_sems.at[i])
            for i, (src, dst) in enumerate([
                (rrow_hbm, rrow_vm), (x_hbm, x_vm), (rW_hbm, rW_vm),
                (eW_hbm, eW_vm), (sW_hbm, sW_vm)])
        ]
        for f in fetches:
            f.start()

        barrier_sem = pltpu.get_barrier_semaphore()
        for nbr in range(N_DEV):
            pl.semaphore_signal(
                barrier_sem, inc=1,
                device_id=(nbr,), device_id_type=pl.DeviceIdType.MESH,
            )

        fetches[0].wait()
        route = rrow_vm[...]
        owner = route // E_LOCAL
        mine = owner == me
        elv = route - owner * E_LOCAL
        eidx = lax.broadcasted_iota(jnp.int32, (E_LOCAL, T), 0)
        Mi = ((eidx == elv) & mine).astype(jnp.int32)

        acc = Mi
        k = 1
        while k < T:
            acc = acc + jnp.concatenate(
                [jnp.zeros((E_LOCAL, k), jnp.int32), acc[:, :T - k]], axis=1)
            k *= 2
        POS = acc - 1

        POSr = jnp.broadcast_to(POS[:, None, :], (E_LOCAL, CAP, T)
                                ).reshape(NSLOT, T)
        Mr = jnp.broadcast_to(Mi[:, None, :], (E_LOCAL, CAP, T)
                              ).reshape(NSLOT, T)
        c_i = lax.broadcasted_iota(jnp.int32, (NSLOT, T), 0) % CAP
        OHw = ((POSr == c_i) & (Mr > 0)).astype(jnp.float32)

        fetches[1].wait()
        fetches[2].wait()
        scores = jnp.dot(x_vm[...], rW_vm[...],
                         preferred_element_type=jnp.float32)
        smax = jnp.max(scores, axis=-1, keepdims=True)
        w_full = 1.0 / jnp.sum(jnp.exp(scores - smax), axis=-1,
                               keepdims=True)

        t_i = lax.broadcasted_iota(jnp.int32, (T, 2), 0)
        dig = jnp.where(lax.broadcasted_iota(jnp.int32, (T, 2), 1) == 0,
                        t_i // 8, t_i % 8).astype(jnp.float32)
        idd = jnp.dot(OHw, jnp.concatenate([dig, w_full], axis=1),
                      preferred_element_type=jnp.float32)
        ids = (8.0 * idd[:, :1] + idd[:, 1:2]).astype(jnp.int32)
        w_sel = idd[:, 2:]
        x_sel = jnp.dot(OHw, x_vm[...],
                        preferred_element_type=jnp.float32) * w_sel

        dev = ids // TPB
        meta_vm[...] = jnp.concatenate([dev, ids - dev * TPB], axis=1)
        n_vm[...] = jnp.minimum(jnp.sum(Mi, axis=1, keepdims=True), CAP)

        cp1 = pltpu.make_async_copy(meta_vm, meta_sm, prep_sem)
        cp1.start()
        cp2 = pltpu.make_async_copy(n_vm, n_sm, prep_sem)
        cp2.start()
        cp1.wait()
        cp2.wait()

        pl.semaphore_wait(barrier_sem, N_DEV)

        fetches[3].wait()
        for el in range(E_LOCAL):
            y_buf[el] = jnp.dot(x_sel[el * CAP:(el + 1) * CAP, :], eW_vm[el],
                                preferred_element_type=jnp.float32)

            def send_body(c, carry, el=el):
                pltpu.make_async_remote_copy(
                    src_ref=y_buf.at[el, pl.ds(c, 1)],
                    dst_ref=recv_buf.at[pl.ds(meta_sm[el * CAP + c, 1], 1)],
                    send_sem=send_sem,
                    recv_sem=recv_sem,
                    device_id=(meta_sm[el * CAP + c, 0],),
                    device_id_type=pl.DeviceIdType.MESH,
                ).start()
                return carry
            lax.fori_loop(0, n_sm[el, 0], send_body, 0)

        fetches[4].wait()
        x_blk = x_vm[pl.ds(me * TPB, TPB), :]
        shared = jnp.dot(x_blk, sW_vm[...],
                         preferred_element_type=jnp.float32)

        for el in range(E_LOCAL):
            def drain_body(c, carry, el=el):
                pltpu.make_async_remote_copy(
                    src_ref=y_buf.at[el, pl.ds(c, 1)],
                    dst_ref=recv_buf.at[pl.ds(0, 1)],
                    send_sem=send_sem,
                    recv_sem=recv_sem,
                    device_id=(0,),
                    device_id_type=pl.DeviceIdType.MESH,
                ).wait_send()
                return carry
            lax.fori_loop(0, n_sm[el, 0], drain_body, 0)

        pltpu.make_async_remote_copy(
            src_ref=y_buf.at[0],
            dst_ref=recv_buf,
            send_sem=send_sem,
            recv_sem=recv_sem,
            device_id=(0,),
            device_id_type=pl.DeviceIdType.MESH,
        ).wait_recv()

        out_ref[...] = shared + recv_buf[...]

    return pl.pallas_call(
        body,
        out_shape=jax.ShapeDtypeStruct((TPB, H), jnp.float32),
        in_specs=[pl.BlockSpec(memory_space=pl.ANY)] * 5,
        out_specs=pl.BlockSpec(memory_space=pltpu.VMEM),
        scratch_shapes=[
            pltpu.VMEM((T, D), jnp.float32),
            pltpu.VMEM((D, E), jnp.float32),
            pltpu.VMEM((1, T), jnp.int32),
            pltpu.VMEM((E_LOCAL, D, H), jnp.float32),
            pltpu.VMEM((D, H), jnp.float32),
            pltpu.VMEM((E_LOCAL, CAP, H), jnp.float32),
            pltpu.VMEM((TPB, H), jnp.float32),
            pltpu.VMEM((NSLOT, 2), jnp.int32),
            pltpu.VMEM((E_LOCAL, 1), jnp.int32),
            pltpu.SMEM((NSLOT, 2), jnp.int32),
            pltpu.SMEM((E_LOCAL, 1), jnp.int32),
            pltpu.SemaphoreType.DMA((5,)),
            pltpu.SemaphoreType.DMA,
            pltpu.SemaphoreType.DMA,
            pltpu.SemaphoreType.DMA,
        ],
        compiler_params=pltpu.CompilerParams(collective_id=0),
    )(x, router_W, route_row, expert_W, shared_W)


# device time: 19421 ns/iter; 1.0510x vs baseline; 1.0510x over previous
import jax
import jax.numpy as jnp
from jax import lax
from jax.experimental import pallas as pl
from jax.experimental.pallas import tpu as pltpu

N_DEV = 32
T = 1024
D = 256
E = 128
H = 512
E_LOCAL = E // N_DEV
TPB = T // N_DEV
CAP = 32
NSLOT = E_LOCAL * CAP


def kernel(x, router_W, route_idx, expert_W, shared_W):
    route_row = route_idx.T

    def body(x_ref, rW_ref, rrow_ref, eW_ref, sW_ref, out_ref,
             y_buf, recv_buf, meta_vm, n_vm, meta_sm, n_sm,
             prep_sem, send_sem, recv_sem):
        me = lax.axis_index("i")

        barrier_sem = pltpu.get_barrier_semaphore()
        for nbr in range(N_DEV):
            pl.semaphore_signal(
                barrier_sem, inc=1,
                device_id=(nbr,), device_id_type=pl.DeviceIdType.MESH,
            )

        route = rrow_ref[...]
        owner = route // E_LOCAL
        mine = owner == me
        elv = route - owner * E_LOCAL
        eidx = lax.broadcasted_iota(jnp.int32, (E_LOCAL, T), 0)
        Mi = ((eidx == elv) & mine).astype(jnp.int32)

        acc = Mi
        k = 1
        while k < T:
            acc = acc + jnp.concatenate(
                [jnp.zeros((E_LOCAL, k), jnp.int32), acc[:, :T - k]], axis=1)
            k *= 2
        POS = acc - 1

        POSr = jnp.broadcast_to(POS[:, None, :], (E_LOCAL, CAP, T)
                                ).reshape(NSLOT, T)
        Mr = jnp.broadcast_to(Mi[:, None, :], (E_LOCAL, CAP, T)
                              ).reshape(NSLOT, T)
        c_i = lax.broadcasted_iota(jnp.int32, (NSLOT, T), 0) % CAP
        OHw = ((POSr == c_i) & (Mr > 0)).astype(jnp.float32)

        scores = jnp.dot(x_ref[...], rW_ref[...],
                         preferred_element_type=jnp.float32)
        smax = jnp.max(scores, axis=-1, keepdims=True)
        w_full = 1.0 / jnp.sum(jnp.exp(scores - smax), axis=-1,
                               keepdims=True)

        t_i = lax.broadcasted_iota(jnp.int32, (T, 2), 0)
        dig = jnp.where(lax.broadcasted_iota(jnp.int32, (T, 2), 1) == 0,
                        t_i // 8, t_i % 8).astype(jnp.float32)
        idd = jnp.dot(OHw, jnp.concatenate([dig, w_full], axis=1),
                      preferred_element_type=jnp.float32)
        ids = (8.0 * idd[:, :1] + idd[:, 1:2]).astype(jnp.int32)
        w_sel = idd[:, 2:]
        x_sel = jnp.dot(OHw, x_ref[...],
                        preferred_element_type=jnp.float32) * w_sel

        dev = ids // TPB
        meta_vm[...] = jnp.concatenate([dev, ids - dev * TPB], axis=1)
        n_vm[...] = jnp.minimum(jnp.sum(Mi, axis=1, keepdims=True), CAP)

        cp1 = pltpu.make_async_copy(meta_vm, meta_sm, prep_sem)
        cp1.start()
        cp2 = pltpu.make_async_copy(n_vm, n_sm, prep_sem)
        cp2.start()
        cp1.wait()
        cp2.wait()

        pl.semaphore_wait(barrier_sem, N_DEV)

        for el in range(E_LOCAL):
            y_buf[el] = jnp.dot(x_sel[el * CAP:(el + 1) * CAP, :], eW_ref[el],
                                preferred_element_type=jnp.float32)

            def send_body(c, carry, el=el):
                pltpu.make_async_remote_copy(
                    src_ref=y_buf.at[el, pl.ds(c, 1)],
                    dst_ref=recv_buf.at[pl.ds(meta_sm[el * CAP + c, 1], 1)],
                    send_sem=send_sem,
                    recv_sem=recv_sem,
                    device_id=(meta_sm[el * CAP + c, 0],),
                    device_id_type=pl.DeviceIdType.MESH,
                ).start()
                return carry
            lax.fori_loop(0, n_sm[el, 0], send_body, 0)

        x_blk = x_ref[pl.ds(me * TPB, TPB), :]
        shared = jnp.dot(x_blk, sW_ref[...],
                         preferred_element_type=jnp.float32)

        for el in range(E_LOCAL):
            def drain_body(c, carry, el=el):
                pltpu.make_async_remote_copy(
                    src_ref=y_buf.at[el, pl.ds(c, 1)],
                    dst_ref=recv_buf.at[pl.ds(0, 1)],
                    send_sem=send_sem,
                    recv_sem=recv_sem,
                    device_id=(0,),
                    device_id_type=pl.DeviceIdType.MESH,
                ).wait_send()
                return carry
            lax.fori_loop(0, n_sm[el, 0], drain_body, 0)

        pltpu.make_async_remote_copy(
            src_ref=y_buf.at[0],
            dst_ref=recv_buf,
            send_sem=send_sem,
            recv_sem=recv_sem,
            device_id=(0,),
            device_id_type=pl.DeviceIdType.MESH,
        ).wait_recv()

        out_ref[...] = shared + recv_buf[...]

    return pl.pallas_call(
        body,
        out_shape=jax.ShapeDtypeStruct((TPB, H), jnp.float32),
        in_specs=[pl.BlockSpec(memory_space=pltpu.VMEM)] * 5,
        out_specs=pl.BlockSpec(memory_space=pltpu.VMEM),
        scratch_shapes=[
            pltpu.VMEM((E_LOCAL, CAP, H), jnp.float32),
            pltpu.VMEM((TPB, H), jnp.float32),
            pltpu.VMEM((NSLOT, 2), jnp.int32),
            pltpu.VMEM((E_LOCAL, 1), jnp.int32),
            pltpu.SMEM((NSLOT, 2), jnp.int32),
            pltpu.SMEM((E_LOCAL, 1), jnp.int32),
            pltpu.SemaphoreType.DMA,
            pltpu.SemaphoreType.DMA,
            pltpu.SemaphoreType.DMA,
        ],
        compiler_params=pltpu.CompilerParams(collective_id=0),
    )(x, router_W, route_row, expert_W, shared_W)
